# baseline (speedup 1.0000x reference)
import functools

import jax
import jax.numpy as jnp
from jax import lax
from jax.experimental import pallas as pl
from jax.experimental.pallas import tpu as pltpu
from jax.experimental.pallas import tpu_sc as plsc

NC = 2
NS = 16
NW = NC * NS
LANE = 16
CHUNK = 128


def _zero_fill(ref, nrows, d):
    z = jnp.zeros((LANE,), jnp.float32)
    for r in range(nrows):
        for c in range(d // LANE):
            ref[r, pl.ds(c * LANE, LANE)] = z


def _zero_acc(acc, sid, n_per_tile, zbuf):
    zr = zbuf.shape[0]
    _zero_fill(zbuf, zr, zbuf.shape[1])
    base = sid * n_per_tile
    nfull = n_per_tile // zr

    def body(i, _):
        pltpu.sync_copy(zbuf, acc.at[pl.ds(base + i * zr, zr)])
        return _

    lax.fori_loop(0, nfull, body, None)
    rem = n_per_tile - nfull * zr
    if rem:
        pltpu.sync_copy(zbuf.at[pl.ds(0, rem)],
                        acc.at[pl.ds(base + nfull * zr, rem)])


@functools.lru_cache(maxsize=None)
def _make_deg(n_acc, nrows, ch):
    npt = n_acc // NS
    mesh = plsc.VectorSubcoreMesh(core_axis_name="c", subcore_axis_name="s")

    @functools.partial(
        pl.kernel,
        out_type=jax.ShapeDtypeStruct((NC, n_acc, LANE), jnp.float32),
        mesh=mesh,
        compiler_params=pltpu.CompilerParams(use_tc_tiling_on_sc=False),
        scratch_types=[
            pltpu.VMEM((2, nrows, ch), jnp.int32),
            pltpu.VMEM((ch, LANE), jnp.float32),
            pltpu.VMEM((4, LANE), jnp.float32),
            pltpu.VMEM_SHARED((n_acc, LANE), jnp.float32),
        ],
    )
    def deg_kernel(edg_hbm, out_hbm, edgbuf, ones_v, zbuf, acc):
        c = lax.axis_index("c")
        s = lax.axis_index("s")
        wid = c * NS + s

        _zero_acc(acc, s, npt, zbuf)

        one = jnp.full((LANE,), 1.0, jnp.float32)
        for r in range(ch):
            ones_v[r] = one

        pltpu.sync_copy(edg_hbm.at[wid], edgbuf)

        plsc.subcore_barrier()

        def scatter(j, _):
            pltpu.sync_copy(ones_v, acc.at[edgbuf.at[1, j]], add=True)
            return _

        lax.fori_loop(0, nrows, scatter, None)

        plsc.subcore_barrier()
        pltpu.sync_copy(acc.at[pl.ds(s * npt, npt)],
                        out_hbm.at[c, pl.ds(s * npt, npt)])

    return deg_kernel


@functools.lru_cache(maxsize=None)
def _make_agg(n, n_acc, nrows, ch, d):
    npt = n_acc // NS
    mesh = plsc.VectorSubcoreMesh(core_axis_name="c", subcore_axis_name="s")

    nbuf = 2 if d >= 128 else 4
    nph = 4 if d >= 128 else 1
    assert nrows % (nph * nbuf) == 0 and nrows // (nph * nbuf) >= 2
    nrp = nrows // nph

    @functools.partial(
        pl.kernel,
        out_type=jax.ShapeDtypeStruct((NC, n_acc, d), jnp.float32),
        mesh=mesh,
        compiler_params=pltpu.CompilerParams(use_tc_tiling_on_sc=False),
        scratch_types=[
            pltpu.VMEM((2, nrp, ch), jnp.int32),
            pltpu.VMEM((4, d), jnp.float32),
            pltpu.VMEM_SHARED((n_acc, d), jnp.float32),
        ] + [pltpu.VMEM((ch, d), jnp.float32)] * nbuf
          + [pltpu.SemaphoreType.DMA] * nbuf,
    )
    def agg_kernel(hs_hbm, edg_hbm, out_hbm, edgbuf, zbuf, acc, *rest):
        rbufs = rest[:nbuf]
        sems = rest[nbuf:]
        c = lax.axis_index("c")
        s = lax.axis_index("s")
        wid = c * NS + s

        _zero_acc(acc, s, npt, zbuf)

        def gfire(j, b):
            return pltpu.async_copy(hs_hbm.at[edgbuf.at[0, j]],
                                    rbufs[b], sems[b])

        def scat(j, b):
            pltpu.sync_copy(rbufs[b], acc.at[edgbuf.at[1, j]], add=True)

        first = True
        for ph in range(nph):
            pltpu.sync_copy(edg_hbm.at[wid, :, pl.ds(ph * nrp, nrp)],
                            edgbuf)
            if first:
                plsc.subcore_barrier()
                first = False

            def batch(g, _):
                j0 = g * nbuf
                handles = [gfire(j0 + b, b) for b in range(nbuf)]
                for b in range(nbuf):
                    handles[b].wait()
                    scat(j0 + b, b)
                return _

            lax.fori_loop(0, nrp // nbuf, batch, None)

        plsc.subcore_barrier()
        pltpu.sync_copy(acc.at[pl.ds(s * npt, npt)],
                        out_hbm.at[c, pl.ds(s * npt, npt)])

    return agg_kernel


def _dinv_block(degp):
    deg = jnp.sum(degp[0] + degp[1], axis=1, keepdims=True) * (1.0 / LANE)
    return lax.rsqrt(deg + 1.0)


def _k1_body(degp_ref, x_ref, w1_ref, hs_ref):
    dinv = _dinv_block(degp_ref[...])
    h = lax.dot_general(x_ref[...], w1_ref[...], (((1,), (1,)), ((), ())),
                        preferred_element_type=jnp.float32)
    hs_ref[...] = h * dinv


def _k2_body(degp_ref, aggp_ref, hs1_ref, b1_ref, w3_ref, hs2_ref):
    dinv = _dinv_block(degp_ref[...])
    aggp = aggp_ref[...]
    t = (aggp[0] + aggp[1] + hs1_ref[...]) * dinv + b1_ref[...]
    h1 = jnp.maximum(t, 0.0)
    h2 = lax.dot_general(h1, w3_ref[...], (((1,), (1,)), ((), ())),
                         preferred_element_type=jnp.float32)
    hs2_ref[...] = h2 * dinv


def _k3_body(degp_ref, aggp_ref, hs2_ref, b3_ref, z_ref):
    dinv = _dinv_block(degp_ref[...])
    aggp = aggp_ref[...]
    z_ref[...] = (aggp[0] + aggp[1] + hs2_ref[...]) * dinv + b3_ref[...]


def kernel(x, edge_index, W1, b1, W3, b3):
    n, d_in = x.shape
    e = edge_index.shape[1]
    d_hid = W1.shape[0]
    d_out = W3.shape[0]
    assert n % NS == 0
    ch = CHUNK
    nrows = -(-e // (NW * ch))
    nrows = max(-(-nrows // 4) * 4, 8)
    e_pad = NW * nrows * ch
    n_acc = -(-(n + 1) // (NS * 8)) * (NS * 8)

    pad = e_pad - e
    dummy_dst = n + jnp.arange(pad, dtype=jnp.int32) % (n_acc - n)
    src3d = jnp.concatenate(
        [edge_index[0], jnp.zeros((pad,), jnp.int32)]).reshape(NW, nrows, ch)
    dst3d = jnp.concatenate(
        [edge_index[1], dummy_dst]).reshape(NW, nrows, ch)
    edges4d = jnp.stack([src3d, dst3d], axis=1)

    rb = 1000
    grid = n // rb

    degp = _make_deg(n_acc, nrows, ch)(edges4d)

    hs1 = pl.pallas_call(
        _k1_body,
        grid=(grid,),
        in_specs=[
            pl.BlockSpec((NC, rb, LANE), lambda i: (0, i, 0)),
            pl.BlockSpec((rb, d_in), lambda i: (i, 0)),
            pl.BlockSpec((d_hid, d_in), lambda i: (0, 0)),
        ],
        out_specs=pl.BlockSpec((rb, d_hid), lambda i: (i, 0)),
        out_shape=jax.ShapeDtypeStruct((n, d_hid), jnp.float32),
    )(degp, x, W1)

    aggp1 = _make_agg(n, n_acc, nrows, ch, d_hid)(hs1, edges4d)

    hs2 = pl.pallas_call(
        _k2_body,
        grid=(grid,),
        in_specs=[
            pl.BlockSpec((NC, rb, LANE), lambda i: (0, i, 0)),
            pl.BlockSpec((NC, rb, d_hid), lambda i: (0, i, 0)),
            pl.BlockSpec((rb, d_hid), lambda i: (i, 0)),
            pl.BlockSpec((1, d_hid), lambda i: (0, 0)),
            pl.BlockSpec((d_out, d_hid), lambda i: (0, 0)),
        ],
        out_specs=pl.BlockSpec((rb, d_out), lambda i: (i, 0)),
        out_shape=jax.ShapeDtypeStruct((n, d_out), jnp.float32),
    )(degp, aggp1, hs1, b1.reshape(1, d_hid), W3)

    aggp2 = _make_agg(n, n_acc, nrows, ch, d_out)(hs2, edges4d)

    z = pl.pallas_call(
        _k3_body,
        grid=(grid,),
        in_specs=[
            pl.BlockSpec((NC, rb, LANE), lambda i: (0, i, 0)),
            pl.BlockSpec((NC, rb, d_out), lambda i: (0, i, 0)),
            pl.BlockSpec((rb, d_out), lambda i: (i, 0)),
            pl.BlockSpec((1, d_out), lambda i: (0, 0)),
        ],
        out_specs=pl.BlockSpec((rb, d_out), lambda i: (i, 0)),
        out_shape=jax.ShapeDtypeStruct((n, d_out), jnp.float32),
    )(degp, aggp2, hs2, b3.reshape(1, d_out))

    return z


# speedup vs baseline: 1.4863x; 1.4863x over previous
import functools

import jax
import jax.numpy as jnp
from jax import lax
from jax.experimental import pallas as pl
from jax.experimental.pallas import tpu as pltpu
from jax.experimental.pallas import tpu_sc as plsc

NC = 2
NS = 16
NW = NC * NS
LANE = 16
CHUNK = 128


def _zero_fill(ref, nrows, d):
    z = jnp.zeros((LANE,), jnp.float32)
    for r in range(nrows):
        for c in range(d // LANE):
            ref[r, pl.ds(c * LANE, LANE)] = z


def _zero_acc(acc, sid, n_per_tile, zbuf):
    zr = zbuf.shape[0]
    _zero_fill(zbuf, zr, zbuf.shape[1])
    base = sid * n_per_tile
    nfull = n_per_tile // zr

    def body(i, _):
        pltpu.sync_copy(zbuf, acc.at[pl.ds(base + i * zr, zr)])
        return _

    lax.fori_loop(0, nfull, body, None)
    rem = n_per_tile - nfull * zr
    if rem:
        pltpu.sync_copy(zbuf.at[pl.ds(0, rem)],
                        acc.at[pl.ds(base + nfull * zr, rem)])


@functools.lru_cache(maxsize=None)
def _make_deg(n_acc, rpw):
    npt = n_acc // NS
    mesh = plsc.VectorSubcoreMesh(core_axis_name="c", subcore_axis_name="s")

    @functools.partial(
        pl.kernel,
        out_type=jax.ShapeDtypeStruct((NC, n_acc, LANE), jnp.float32),
        mesh=mesh,
        compiler_params=pltpu.CompilerParams(use_tc_tiling_on_sc=False),
        scratch_types=[
            pltpu.VMEM((rpw, CHUNK), jnp.int32),
            pltpu.VMEM((CHUNK, LANE), jnp.float32),
            pltpu.VMEM((8, LANE), jnp.float32),
            pltpu.VMEM_SHARED((n_acc, LANE), jnp.float32),
        ],
    )
    def deg_kernel(dst_hbm, out_hbm, dstbuf, ones_v, zbuf, acc):
        c = lax.axis_index("c")
        s = lax.axis_index("s")
        wid = c * NS + s

        _zero_acc(acc, s, npt, zbuf)

        one = jnp.full((LANE,), 1.0, jnp.float32)
        for r in range(CHUNK):
            ones_v[r] = one

        pltpu.sync_copy(dst_hbm.at[wid], dstbuf)

        plsc.subcore_barrier()

        def scatter(j, _):
            pltpu.sync_copy(ones_v, acc.at[dstbuf.at[j]], add=True)
            return _

        lax.fori_loop(0, rpw, scatter, None)

        plsc.subcore_barrier()
        pltpu.sync_copy(acc.at[pl.ds(s * npt, npt)],
                        out_hbm.at[c, pl.ds(s * npt, npt)])

    return deg_kernel


@functools.lru_cache(maxsize=None)
def _make_agg(n, n_acc, rpw, d):
    npt = n_acc // NS
    mesh = plsc.VectorSubcoreMesh(core_axis_name="c", subcore_axis_name="s")

    @functools.partial(
        pl.kernel,
        out_type=jax.ShapeDtypeStruct((NC, n_acc, d), jnp.float32),
        mesh=mesh,
        compiler_params=pltpu.CompilerParams(
            use_tc_tiling_on_sc=(d % 128 == 0)),
        scratch_types=[
            pltpu.VMEM((rpw, CHUNK), jnp.int32),
            pltpu.VMEM((rpw, CHUNK), jnp.int32),
            pltpu.VMEM((CHUNK, d), jnp.float32),
            pltpu.VMEM((8, d), jnp.float32),
            pltpu.VMEM_SHARED((n_acc, d), jnp.float32),
            pltpu.SemaphoreType.DMA,
        ],
    )
    def agg_kernel(hs_hbm, src_hbm, dst_hbm, out_hbm,
                   srcbuf, dstbuf, rbuf, zbuf, acc, sem):
        c = lax.axis_index("c")
        s = lax.axis_index("s")
        wid = c * NS + s

        _zero_acc(acc, s, npt, zbuf)

        pltpu.sync_copy(src_hbm.at[wid], srcbuf)
        pltpu.sync_copy(dst_hbm.at[wid], dstbuf)

        plsc.subcore_barrier()

        def step(j, _):
            pltpu.async_copy(hs_hbm.at[srcbuf.at[j]], rbuf, sem).wait()
            pltpu.sync_copy(rbuf, acc.at[dstbuf.at[j]], add=True)
            return _

        lax.fori_loop(0, rpw, step, None)

        plsc.subcore_barrier()
        pltpu.sync_copy(acc.at[pl.ds(s * npt, npt)],
                        out_hbm.at[c, pl.ds(s * npt, npt)])

    return agg_kernel


def _dinv_block(degp):
    deg = jnp.sum(degp[0] + degp[1], axis=1, keepdims=True) * (1.0 / LANE)
    return lax.rsqrt(deg + 1.0)


def _k1_body(degp_ref, x_ref, w1_ref, hs_ref):
    dinv = _dinv_block(degp_ref[...])
    h = lax.dot_general(x_ref[...], w1_ref[...], (((1,), (1,)), ((), ())),
                        preferred_element_type=jnp.float32)
    hs_ref[...] = h * dinv


def _k2_body(degp_ref, aggp_ref, hs1_ref, b1_ref, w3_ref, hs2_ref):
    dinv = _dinv_block(degp_ref[...])
    aggp = aggp_ref[...]
    t = (aggp[0] + aggp[1] + hs1_ref[...]) * dinv + b1_ref[...]
    h1 = jnp.maximum(t, 0.0)
    h2 = lax.dot_general(h1, w3_ref[...], (((1,), (1,)), ((), ())),
                         preferred_element_type=jnp.float32)
    hs2_ref[...] = h2 * dinv


def _k3_body(degp_ref, aggp_ref, hs2_ref, b3_ref, z_ref):
    dinv = _dinv_block(degp_ref[...])
    aggp = aggp_ref[...]
    z_ref[...] = (aggp[0] + aggp[1] + hs2_ref[...]) * dinv + b3_ref[...]


def kernel(x, edge_index, W1, b1, W3, b3):
    n, d_in = x.shape
    e = edge_index.shape[1]
    d_hid = W1.shape[0]
    d_out = W3.shape[0]
    assert n % NS == 0
    rpw = -(-e // (NW * CHUNK))
    e_pad = NW * rpw * CHUNK
    n_acc = -(-(n + 1) // (NS * 8)) * (NS * 8)

    pad = e_pad - e
    dummy_dst = n + jnp.arange(pad, dtype=jnp.int32) % (n_acc - n)
    src3d = jnp.concatenate(
        [edge_index[0], jnp.zeros((pad,), jnp.int32)]).reshape(NW, rpw, CHUNK)
    dst3d = jnp.concatenate(
        [edge_index[1], dummy_dst]).reshape(NW, rpw, CHUNK)

    rb = 1000
    grid = n // rb

    degp = _make_deg(n_acc, rpw)(dst3d)

    hs1 = pl.pallas_call(
        _k1_body,
        grid=(grid,),
        in_specs=[
            pl.BlockSpec((NC, rb, LANE), lambda i: (0, i, 0)),
            pl.BlockSpec((rb, d_in), lambda i: (i, 0)),
            pl.BlockSpec((d_hid, d_in), lambda i: (0, 0)),
        ],
        out_specs=pl.BlockSpec((rb, d_hid), lambda i: (i, 0)),
        out_shape=jax.ShapeDtypeStruct((n, d_hid), jnp.float32),
    )(degp, x, W1)

    aggp1 = _make_agg(n, n_acc, rpw, d_hid)(hs1, src3d, dst3d)

    hs2 = pl.pallas_call(
        _k2_body,
        grid=(grid,),
        in_specs=[
            pl.BlockSpec((NC, rb, LANE), lambda i: (0, i, 0)),
            pl.BlockSpec((NC, rb, d_hid), lambda i: (0, i, 0)),
            pl.BlockSpec((rb, d_hid), lambda i: (i, 0)),
            pl.BlockSpec((1, d_hid), lambda i: (0, 0)),
            pl.BlockSpec((d_out, d_hid), lambda i: (0, 0)),
        ],
        out_specs=pl.BlockSpec((rb, d_out), lambda i: (i, 0)),
        out_shape=jax.ShapeDtypeStruct((n, d_out), jnp.float32),
    )(degp, aggp1, hs1, b1.reshape(1, d_hid), W3)

    aggp2 = _make_agg(n, n_acc, rpw, d_out)(hs2, src3d, dst3d)

    z = pl.pallas_call(
        _k3_body,
        grid=(grid,),
        in_specs=[
            pl.BlockSpec((NC, rb, LANE), lambda i: (0, i, 0)),
            pl.BlockSpec((NC, rb, d_out), lambda i: (0, i, 0)),
            pl.BlockSpec((rb, d_out), lambda i: (i, 0)),
            pl.BlockSpec((1, d_out), lambda i: (0, 0)),
        ],
        out_specs=pl.BlockSpec((rb, d_out), lambda i: (i, 0)),
        out_shape=jax.ShapeDtypeStruct((n, d_out), jnp.float32),
    )(degp, aggp2, hs2, b3.reshape(1, d_out))

    return z


# speedup vs baseline: 1.5409x; 1.0368x over previous
import functools

import jax
import jax.numpy as jnp
from jax import lax
from jax.experimental import pallas as pl
from jax.experimental.pallas import tpu as pltpu
from jax.experimental.pallas import tpu_sc as plsc

NC = 2
NS = 16
NW = NC * NS
LANE = 16
CHUNK = 128


def _zero_fill(ref, nrows, d):
    z = jnp.zeros((LANE,), jnp.float32)
    for r in range(nrows):
        for c in range(d // LANE):
            ref[r, pl.ds(c * LANE, LANE)] = z


def _zero_acc(acc, sid, n_per_tile, zbuf):
    zr = zbuf.shape[0]
    _zero_fill(zbuf, zr, zbuf.shape[1])
    base = sid * n_per_tile
    nfull = n_per_tile // zr

    def body(i, _):
        pltpu.sync_copy(zbuf, acc.at[pl.ds(base + i * zr, zr)])
        return _

    lax.fori_loop(0, nfull, body, None)
    rem = n_per_tile - nfull * zr
    if rem:
        pltpu.sync_copy(zbuf.at[pl.ds(0, rem)],
                        acc.at[pl.ds(base + nfull * zr, rem)])


@functools.lru_cache(maxsize=None)
def _make_deg(n_acc, rpw):
    npt = n_acc // NS
    mesh = plsc.VectorSubcoreMesh(core_axis_name="c", subcore_axis_name="s")

    @functools.partial(
        pl.kernel,
        out_type=jax.ShapeDtypeStruct((NC, n_acc, LANE), jnp.float32),
        mesh=mesh,
        compiler_params=pltpu.CompilerParams(use_tc_tiling_on_sc=False),
        scratch_types=[
            pltpu.VMEM((rpw, CHUNK), jnp.int32),
            pltpu.VMEM((CHUNK, LANE), jnp.float32),
            pltpu.VMEM((8, LANE), jnp.float32),
            pltpu.VMEM_SHARED((n_acc, LANE), jnp.float32),
        ],
    )
    def deg_kernel(dst_hbm, out_hbm, dstbuf, ones_v, zbuf, acc):
        c = lax.axis_index("c")
        s = lax.axis_index("s")
        wid = c * NS + s

        _zero_acc(acc, s, npt, zbuf)

        one = jnp.full((LANE,), 1.0, jnp.float32)
        for r in range(CHUNK):
            ones_v[r] = one

        pltpu.sync_copy(dst_hbm.at[wid], dstbuf)

        plsc.subcore_barrier()

        def scatter(j, _):
            pltpu.sync_copy(ones_v, acc.at[dstbuf.at[j]], add=True)
            return _

        lax.fori_loop(0, rpw, scatter, None)

        plsc.subcore_barrier()
        pltpu.sync_copy(acc.at[pl.ds(s * npt, npt)],
                        out_hbm.at[c, pl.ds(s * npt, npt)])

    return deg_kernel


@functools.lru_cache(maxsize=None)
def _make_agg(n, n_acc, rpw, d):
    npt = n_acc // NS
    mesh = plsc.VectorSubcoreMesh(core_axis_name="c", subcore_axis_name="s")

    nbuf = 2 if d <= 64 else 1

    scratch = [
        pltpu.VMEM((rpw, CHUNK), jnp.int32),
        pltpu.VMEM((rpw, CHUNK), jnp.int32),
    ]
    scratch += [pltpu.VMEM((CHUNK, d), jnp.float32)] * nbuf
    scratch += [
        pltpu.VMEM((8, d), jnp.float32),
        pltpu.VMEM_SHARED((n_acc, d), jnp.float32),
    ]
    scratch += [pltpu.SemaphoreType.DMA] * nbuf

    @functools.partial(
        pl.kernel,
        out_type=jax.ShapeDtypeStruct((NC, n_acc, d), jnp.float32),
        mesh=mesh,
        compiler_params=pltpu.CompilerParams(
            use_tc_tiling_on_sc=(d % 128 == 0)),
        scratch_types=scratch,
    )
    def agg_kernel(hs_hbm, src_hbm, dst_hbm, out_hbm, *refs):
        if nbuf == 2:
            srcbuf, dstbuf, rbuf0, rbuf1, zbuf, acc, sem0, sem1 = refs
        else:
            srcbuf, dstbuf, rbuf0, zbuf, acc, sem0 = refs
        c = lax.axis_index("c")
        s = lax.axis_index("s")
        wid = c * NS + s

        _zero_acc(acc, s, npt, zbuf)

        pltpu.sync_copy(src_hbm.at[wid], srcbuf)
        pltpu.sync_copy(dst_hbm.at[wid], dstbuf)

        plsc.subcore_barrier()

        if nbuf == 2:
            def step(p, _):
                j0 = 2 * p
                j1 = j0 + 1
                c0 = pltpu.async_copy(hs_hbm.at[srcbuf.at[j0]], rbuf0, sem0)
                c1 = pltpu.async_copy(hs_hbm.at[srcbuf.at[j1]], rbuf1, sem1)
                c0.wait()
                pltpu.sync_copy(rbuf0, acc.at[dstbuf.at[j0]], add=True)
                c1.wait()
                pltpu.sync_copy(rbuf1, acc.at[dstbuf.at[j1]], add=True)
                return _

            lax.fori_loop(0, rpw // 2, step, None)
            if rpw % 2:
                j = rpw - 1
                pltpu.async_copy(hs_hbm.at[srcbuf.at[j]], rbuf0, sem0).wait()
                pltpu.sync_copy(rbuf0, acc.at[dstbuf.at[j]], add=True)
        else:
            def step(j, _):
                pltpu.async_copy(hs_hbm.at[srcbuf.at[j]], rbuf0, sem0).wait()
                pltpu.sync_copy(rbuf0, acc.at[dstbuf.at[j]], add=True)
                return _

            lax.fori_loop(0, rpw, step, None)

        plsc.subcore_barrier()
        pltpu.sync_copy(acc.at[pl.ds(s * npt, npt)],
                        out_hbm.at[c, pl.ds(s * npt, npt)])

    return agg_kernel


def _dinv_block(degp):
    deg = jnp.sum(degp[0] + degp[1], axis=1, keepdims=True) * (1.0 / LANE)
    return lax.rsqrt(deg + 1.0)


def _k1_body(degp_ref, x_ref, w1_ref, hs_ref):
    dinv = _dinv_block(degp_ref[...])
    h = lax.dot_general(x_ref[...], w1_ref[...], (((1,), (1,)), ((), ())),
                        preferred_element_type=jnp.float32)
    hs_ref[...] = h * dinv


def _k2_body(degp_ref, aggp_ref, hs1_ref, b1_ref, w3_ref, hs2_ref):
    dinv = _dinv_block(degp_ref[...])
    aggp = aggp_ref[...]
    t = (aggp[0] + aggp[1] + hs1_ref[...]) * dinv + b1_ref[...]
    h1 = jnp.maximum(t, 0.0)
    h2 = lax.dot_general(h1, w3_ref[...], (((1,), (1,)), ((), ())),
                         preferred_element_type=jnp.float32)
    hs2_ref[...] = h2 * dinv


def _k3_body(degp_ref, aggp_ref, hs2_ref, b3_ref, z_ref):
    dinv = _dinv_block(degp_ref[...])
    aggp = aggp_ref[...]
    z_ref[...] = (aggp[0] + aggp[1] + hs2_ref[...]) * dinv + b3_ref[...]


def kernel(x, edge_index, W1, b1, W3, b3):
    n, d_in = x.shape
    e = edge_index.shape[1]
    d_hid = W1.shape[0]
    d_out = W3.shape[0]
    assert n % NS == 0
    rpw = -(-e // (NW * CHUNK))
    e_pad = NW * rpw * CHUNK
    n_acc = -(-(n + 1) // (NS * 8)) * (NS * 8)

    pad = e_pad - e
    dummy_dst = n + jnp.arange(pad, dtype=jnp.int32) % (n_acc - n)
    src3d = jnp.concatenate(
        [edge_index[0], jnp.zeros((pad,), jnp.int32)]).reshape(NW, rpw, CHUNK)
    dst3d = jnp.concatenate(
        [edge_index[1], dummy_dst]).reshape(NW, rpw, CHUNK)

    rb = 1000
    grid = n // rb

    degp = _make_deg(n_acc, rpw)(dst3d)

    hs1 = pl.pallas_call(
        _k1_body,
        grid=(grid,),
        in_specs=[
            pl.BlockSpec((NC, rb, LANE), lambda i: (0, i, 0)),
            pl.BlockSpec((rb, d_in), lambda i: (i, 0)),
            pl.BlockSpec((d_hid, d_in), lambda i: (0, 0)),
        ],
        out_specs=pl.BlockSpec((rb, d_hid), lambda i: (i, 0)),
        out_shape=jax.ShapeDtypeStruct((n, d_hid), jnp.float32),
    )(degp, x, W1)

    aggp1 = _make_agg(n, n_acc, rpw, d_hid)(hs1, src3d, dst3d)

    hs2 = pl.pallas_call(
        _k2_body,
        grid=(grid,),
        in_specs=[
            pl.BlockSpec((NC, rb, LANE), lambda i: (0, i, 0)),
            pl.BlockSpec((NC, rb, d_hid), lambda i: (0, i, 0)),
            pl.BlockSpec((rb, d_hid), lambda i: (i, 0)),
            pl.BlockSpec((1, d_hid), lambda i: (0, 0)),
            pl.BlockSpec((d_out, d_hid), lambda i: (0, 0)),
        ],
        out_specs=pl.BlockSpec((rb, d_out), lambda i: (i, 0)),
        out_shape=jax.ShapeDtypeStruct((n, d_out), jnp.float32),
    )(degp, aggp1, hs1, b1.reshape(1, d_hid), W3)

    aggp2 = _make_agg(n, n_acc, rpw, d_out)(hs2, src3d, dst3d)

    z = pl.pallas_call(
        _k3_body,
        grid=(grid,),
        in_specs=[
            pl.BlockSpec((NC, rb, LANE), lambda i: (0, i, 0)),
            pl.BlockSpec((NC, rb, d_out), lambda i: (0, i, 0)),
            pl.BlockSpec((rb, d_out), lambda i: (i, 0)),
            pl.BlockSpec((1, d_out), lambda i: (0, 0)),
        ],
        out_specs=pl.BlockSpec((rb, d_out), lambda i: (i, 0)),
        out_shape=jax.ShapeDtypeStruct((n, d_out), jnp.float32),
    )(degp, aggp2, hs2, b3.reshape(1, d_out))

    return z
